# baseline (speedup 1.0000x reference)
import jax
import jax.numpy as jnp
from jax import lax
from jax.experimental import pallas as pl
from jax.experimental.pallas import tpu as pltpu
from jax.experimental.pallas import tpu_sc as plsc
from jax.experimental import layout as jax_layout

TOKENS = 8192
HIDDEN = 2048
EXPERTS = 64
TOPK = 8
SCALE = 2.5

BLK = 512
NCHUNK = 2
CHUNK = TOKENS // NCHUNK

_INFO = plsc.get_sparse_core_info()
NC = _INFO.num_cores
NS = _INFO.num_subcores
NW = NC * NS
HR = CHUNK // 2 // NW
TPW = 2 * HR


def _softmax_body(h_ref, w_ref, b_ref, p_ref):
    h = h_ref[...]
    w = w_ref[...]
    logits = lax.dot_general(
        h, w, (((1,), (1,)), ((), ())),
        preferred_element_type=jnp.float32)
    m = jnp.max(logits, axis=-1, keepdims=True)
    e = jnp.exp(logits - m)
    s = jnp.sum(e, axis=-1, keepdims=True)
    res = e / s + b_ref[...]
    p_ref[...] = jnp.concatenate([res[:BLK // 2], res[BLK // 2:]], axis=1)


def _tc_biased_chunk(hidden_states, weight, bias, c):
    nb = CHUNK // BLK
    return pl.pallas_call(
        _softmax_body,
        grid=(nb,),
        in_specs=[
            pl.BlockSpec((BLK, HIDDEN), lambda i, c=c: (c * nb + i, 0)),
            pl.BlockSpec((EXPERTS, HIDDEN), lambda i: (0, 0)),
            pl.BlockSpec((1, EXPERTS), lambda i: (0, 0)),
        ],
        out_specs=pl.BlockSpec((BLK // 2, 2 * EXPERTS), lambda i: (i, 0)),
        out_shape=jax.ShapeDtypeStruct((CHUNK // 2, 2 * EXPERTS),
                                       jnp.float32),
    )(hidden_states, weight, bias)


def _vshift(x, idx):
    dnums = lax.GatherDimensionNumbers(
        offset_dims=(), collapsed_slice_dims=(0,), start_index_map=(0,))
    return lax.gather(x, idx[:, None], dnums, (1,),
                      mode=lax.GatherScatterMode.PROMISE_IN_BOUNDS)


def _sc_topk_body(p_hbm, b_hbm, w_hbm, i_hbm, p_v, b_v, ow_v, oi_v):
    c = lax.axis_index("c")
    s = lax.axis_index("s")
    wid = c * NS + s
    base_row = wid * HR

    pltpu.sync_copy(p_hbm.at[pl.ds(base_row, HR)], p_v)
    pltpu.sync_copy(b_hbm, b_v)

    iota = lax.iota(jnp.int32, 16)
    idx_g = [iota + 16 * v for v in range(4)]
    msk8 = iota < TOPK
    nxt = jnp.minimum(iota + 1, 15)
    prv = jnp.maximum(iota - 1, 0)

    def merge(ak, av, bk, bv):
        rk = lax.rev(bk, (0,))
        rv = lax.rev(bv, (0,))
        ge = ak >= rk
        mk = jnp.where(ge, ak, rk)
        mv = jnp.where(ge, av, rv)
        return plsc.sort_key_val(mk, mv, descending=True)

    def topk_one(row, col, slot):
        sk = []
        sv = []
        for v in range(4):
            k = p_v[row, pl.ds(col + 16 * v, 16)]
            ks, vs = plsc.sort_key_val(k, idx_g[v], descending=True)
            sk.append(ks)
            sv.append(vs)
        m1k, m1v = merge(sk[0], sv[0], sk[1], sv[1])
        m2k, m2v = merge(sk[2], sv[2], sk[3], sv[3])
        fk, fv = merge(m1k, m1v, m2k, m2v)
        kn = _vshift(fk, nxt)
        vn = _vshift(fv, nxt)
        kp = _vshift(fk, prv)
        vp = _vshift(fv, prv)
        nv = jnp.where((fk == kn) & (fv > vn), vn, fv)
        nv = jnp.where((fk == kp) & (vp > fv), vp, nv)
        bg = plsc.load_gather(b_v, [nv])
        wv = (fk - bg) * SCALE
        plsc.store_compressed(ow_v.at[pl.ds(slot, 16)], wv, mask=msk8)
        plsc.store_compressed(oi_v.at[pl.ds(slot, 16)], nv, mask=msk8)

    def rowfn(r, _):
        topk_one(r, 0, r * TOPK)
        topk_one(r, EXPERTS, (HR + r) * TOPK)
        return 0

    lax.fori_loop(0, HR, rowfn, 0)

    g0 = (base_row // (BLK // 2)) * BLK + base_row % (BLK // 2)
    n = HR * TOPK
    pltpu.sync_copy(ow_v.at[pl.ds(0, n)], w_hbm.at[pl.ds(g0 * TOPK, n)])
    pltpu.sync_copy(oi_v.at[pl.ds(0, n)], i_hbm.at[pl.ds(g0 * TOPK, n)])
    g1 = g0 + BLK // 2
    pltpu.sync_copy(ow_v.at[pl.ds(n, n)], w_hbm.at[pl.ds(g1 * TOPK, n)])
    pltpu.sync_copy(oi_v.at[pl.ds(n, n)], i_hbm.at[pl.ds(g1 * TOPK, n)])


_sc_topk = pl.kernel(
    _sc_topk_body,
    out_type=[
        jax.ShapeDtypeStruct((CHUNK * TOPK,), jnp.float32),
        jax.ShapeDtypeStruct((CHUNK * TOPK,), jnp.int32),
    ],
    mesh=plsc.VectorSubcoreMesh(core_axis_name="c", subcore_axis_name="s"),
    compiler_params=pltpu.CompilerParams(needs_layout_passes=False),
    scratch_types=[
        pltpu.VMEM((HR, 2 * EXPERTS), jnp.float32),
        pltpu.VMEM((EXPERTS,), jnp.float32),
        pltpu.VMEM((TPW * TOPK + 16,), jnp.float32),
        pltpu.VMEM((TPW * TOPK + 16,), jnp.int32),
    ],
)


def _kernel_impl(hidden_states, classifier_weight, e_score_correction_bias):
    bias = e_score_correction_bias.reshape(1, EXPERTS)
    ws = []
    inds = []
    for c in range(NCHUNK):
        biased = _tc_biased_chunk(hidden_states, classifier_weight, bias, c)
        w_c, i_c = _sc_topk(biased, e_score_correction_bias)
        ws.append(w_c)
        inds.append(i_c)
    w_flat = jnp.concatenate(ws)
    i_flat = jnp.concatenate(inds)
    return w_flat.reshape(TOKENS, TOPK), i_flat.reshape(TOKENS, TOPK)


_jitted = None


def kernel(hidden_states, classifier_weight, e_score_correction_bias):
    global _jitted
    if _jitted is None:
        dev = jax.devices()[0]
        fmt = jax_layout.Format(
            jax_layout.Layout((0, 1), ()),
            jax.sharding.SingleDeviceSharding(dev))
        _jitted = jax.jit(_kernel_impl, out_shardings=(fmt, fmt))
    return _jitted(hidden_states, classifier_weight,
                   e_score_correction_bias)


# speedup vs baseline: 1.0026x; 1.0026x over previous
import jax
import jax.numpy as jnp
from jax import lax
from jax.experimental import pallas as pl
from jax.experimental.pallas import tpu as pltpu
from jax.experimental.pallas import tpu_sc as plsc

TOKENS = 8192
HIDDEN = 2048
EXPERTS = 64
TOPK = 8
SCALE = 2.5

BLK = 512
NCHUNK = 2
CHUNK = TOKENS // NCHUNK

_INFO = plsc.get_sparse_core_info()
NC = _INFO.num_cores
NS = _INFO.num_subcores
NW = NC * NS
HR = CHUNK // 2 // NW
TPW = 2 * HR


def _softmax_body(h_ref, w_ref, b_ref, p_ref):
    h = h_ref[...]
    w = w_ref[...]
    logits = lax.dot_general(
        h, w, (((1,), (1,)), ((), ())),
        preferred_element_type=jnp.float32)
    m = jnp.max(logits, axis=-1, keepdims=True)
    e = jnp.exp(logits - m)
    s = jnp.sum(e, axis=-1, keepdims=True)
    res = e / s + b_ref[...]
    p_ref[...] = jnp.concatenate([res[:BLK // 2], res[BLK // 2:]], axis=1)


def _tc_biased_chunk(hidden_states, weight, bias, c):
    nb = CHUNK // BLK
    return pl.pallas_call(
        _softmax_body,
        grid=(nb,),
        in_specs=[
            pl.BlockSpec((BLK, HIDDEN), lambda i, c=c: (c * nb + i, 0)),
            pl.BlockSpec((EXPERTS, HIDDEN), lambda i: (0, 0)),
            pl.BlockSpec((1, EXPERTS), lambda i: (0, 0)),
        ],
        out_specs=pl.BlockSpec((BLK // 2, 2 * EXPERTS), lambda i: (i, 0)),
        out_shape=jax.ShapeDtypeStruct((CHUNK // 2, 2 * EXPERTS),
                                       jnp.float32),
    )(hidden_states, weight, bias)


def _vshift(x, idx):
    dnums = lax.GatherDimensionNumbers(
        offset_dims=(), collapsed_slice_dims=(0,), start_index_map=(0,))
    return lax.gather(x, idx[:, None], dnums, (1,),
                      mode=lax.GatherScatterMode.PROMISE_IN_BOUNDS)


def _sc_topk_body(p_hbm, b_hbm, w_hbm, i_hbm, p_v, b_v, ow_v, oi_v):
    c = lax.axis_index("c")
    s = lax.axis_index("s")
    wid = c * NS + s
    base_row = wid * HR

    pltpu.sync_copy(p_hbm.at[pl.ds(base_row, HR)], p_v)
    pltpu.sync_copy(b_hbm, b_v)

    iota = lax.iota(jnp.int32, 16)
    idx_g = [iota + 16 * v for v in range(4)]
    msk8 = iota < TOPK
    nxt = jnp.minimum(iota + 1, 15)
    prv = jnp.maximum(iota - 1, 0)

    def topk_one(row, col, slot):
        sk = []
        sv = []
        for v in range(4):
            k = p_v[row, pl.ds(col + 16 * v, 16)]
            ks, vs = plsc.sort_key_val(k, idx_g[v], descending=(v % 2 == 0))
            sk.append(ks)
            sv.append(vs)
        ge = sk[0] >= sk[1]
        m1k, m1v = plsc.sort_key_val(jnp.where(ge, sk[0], sk[1]),
                                     jnp.where(ge, sv[0], sv[1]),
                                     descending=True)
        ge = sk[2] >= sk[3]
        m2k, m2v = plsc.sort_key_val(jnp.where(ge, sk[2], sk[3]),
                                     jnp.where(ge, sv[2], sv[3]),
                                     descending=False)
        ge = m1k >= m2k
        fk, fv = plsc.sort_key_val(jnp.where(ge, m1k, m2k),
                                   jnp.where(ge, m1v, m2v),
                                   descending=True)
        kn = _vshift(fk, nxt)
        vn = _vshift(fv, nxt)
        kp = _vshift(fk, prv)
        vp = _vshift(fv, prv)
        nv = jnp.where((fk == kn) & (fv > vn), vn, fv)
        nv = jnp.where((fk == kp) & (vp > fv), vp, nv)
        bg = plsc.load_gather(b_v, [nv])
        wv = (fk - bg) * SCALE
        plsc.store_compressed(ow_v.at[pl.ds(slot, 16)], wv, mask=msk8)
        plsc.store_compressed(oi_v.at[pl.ds(slot, 16)], nv, mask=msk8)

    def rowfn(r, _):
        topk_one(r, 0, r * TOPK)
        topk_one(r, EXPERTS, (HR + r) * TOPK)
        return 0

    lax.fori_loop(0, HR, rowfn, 0)

    g0 = (base_row // (BLK // 2)) * BLK + base_row % (BLK // 2)
    n = HR * TOPK
    pltpu.sync_copy(ow_v.at[pl.ds(0, n)], w_hbm.at[pl.ds(g0 * TOPK, n)])
    pltpu.sync_copy(oi_v.at[pl.ds(0, n)], i_hbm.at[pl.ds(g0 * TOPK, n)])
    g1 = g0 + BLK // 2
    pltpu.sync_copy(ow_v.at[pl.ds(n, n)], w_hbm.at[pl.ds(g1 * TOPK, n)])
    pltpu.sync_copy(oi_v.at[pl.ds(n, n)], i_hbm.at[pl.ds(g1 * TOPK, n)])


_sc_topk = pl.kernel(
    _sc_topk_body,
    out_type=[
        jax.ShapeDtypeStruct((CHUNK * TOPK,), jnp.float32),
        jax.ShapeDtypeStruct((CHUNK * TOPK,), jnp.int32),
    ],
    mesh=plsc.VectorSubcoreMesh(core_axis_name="c", subcore_axis_name="s"),
    compiler_params=pltpu.CompilerParams(needs_layout_passes=False),
    scratch_types=[
        pltpu.VMEM((HR, 2 * EXPERTS), jnp.float32),
        pltpu.VMEM((EXPERTS,), jnp.float32),
        pltpu.VMEM((TPW * TOPK + 16,), jnp.float32),
        pltpu.VMEM((TPW * TOPK + 16,), jnp.int32),
    ],
)


def _kernel_impl(hidden_states, classifier_weight, e_score_correction_bias):
    bias = e_score_correction_bias.reshape(1, EXPERTS)
    ws = []
    inds = []
    for c in range(NCHUNK):
        biased = _tc_biased_chunk(hidden_states, classifier_weight, bias, c)
        w_c, i_c = _sc_topk(biased, e_score_correction_bias)
        ws.append(w_c)
        inds.append(i_c)
    w_flat = jnp.concatenate(ws)
    i_flat = jnp.concatenate(inds)
    return w_flat.reshape(TOKENS, TOPK), i_flat.reshape(TOKENS, TOPK)


kernel = jax.jit(_kernel_impl)
